# baseline (speedup 1.0000x reference)
import functools

import jax
import jax.numpy as jnp
from jax import lax
from jax.experimental import pallas as pl
from jax.experimental.pallas import tpu as pltpu
from jax.experimental.pallas import tpu_sc as plsc

VOCAB = 1000000
EMB = 32
BATCH = 16384
BAG = 50

NC, NS = 2, 16
NW = NC * NS
BAGS_PER_W = BATCH // NW
CHUNK = 64
ROWS = CHUNK * BAG
GRP = 128
NGRP = ROWS // GRP
NCHUNK = BAGS_PER_W // CHUNK
HALF = EMB // 2

_HI_MASK = -65536


def _body(idx_hbm, table_hbm, out_hbm, idx_v, rows_v, out_v, sem):
    wid = lax.axis_index("s") * NC + lax.axis_index("c")

    def chunk_body(c, carry):
        pltpu.sync_copy(idx_hbm.at[wid * NCHUNK + c], idx_v)
        copies = [
            pltpu.async_copy(
                table_hbm.at[idx_v.at[g]],
                rows_v.at[pl.ds(g * GRP, GRP)],
                sem,
            )
            for g in range(NGRP)
        ]
        for cp in copies:
            cp.wait()

        def bag_body(b, _):
            base = b * BAG

            def j_body(j, accs):
                ae, ao = accs
                v = rows_v[base + j, :]
                ae = ae + lax.bitcast_convert_type(
                    lax.shift_left(v, 16), jnp.float32)
                ao = ao + lax.bitcast_convert_type(
                    lax.bitwise_and(v, jnp.int32(_HI_MASK)), jnp.float32)
                return ae, ao

            zero = jnp.zeros((HALF,), jnp.float32)
            ae, ao = lax.fori_loop(0, BAG, j_body, (zero, zero), unroll=10)
            out_v[b, 0, :] = ae
            out_v[b, 1, :] = ao
            return _

        lax.fori_loop(0, CHUNK, bag_body, 0)
        pltpu.sync_copy(out_v, out_hbm.at[pl.ds(wid * BAGS_PER_W + c * CHUNK, CHUNK)])
        return carry

    lax.fori_loop(0, NCHUNK, chunk_body, 0)


@jax.jit
def _run(idx_grouped, table_bf):
    mesh = plsc.VectorSubcoreMesh(core_axis_name="c", subcore_axis_name="s")
    f = pl.kernel(
        _body,
        out_type=jax.ShapeDtypeStruct((BATCH, 2, HALF), jnp.float32),
        mesh=mesh,
        scratch_types=[
            pltpu.VMEM((NGRP, GRP), jnp.int32),
            pltpu.VMEM((ROWS, HALF), jnp.int32),
            pltpu.VMEM((CHUNK, 2, HALF), jnp.float32),
            pltpu.SemaphoreType.DMA,
        ],
        compiler_params=pltpu.CompilerParams(use_tc_tiling_on_sc=False),
    )
    return f(idx_grouped, table_bf)


def kernel(indices, table):
    idx_grouped = indices.reshape(NW * NCHUNK, NGRP, GRP)
    table_i = lax.bitcast_convert_type(
        table.astype(jnp.bfloat16).reshape(VOCAB, HALF, 2), jnp.int32)
    planes = _run(idx_grouped, table_i)
    return jnp.swapaxes(planes, 1, 2).reshape(BATCH, EMB)


# speedup vs baseline: 2.0360x; 2.0360x over previous
import functools

import jax
import jax.numpy as jnp
from jax import lax
from jax.experimental import pallas as pl
from jax.experimental.pallas import tpu as pltpu
from jax.experimental.pallas import tpu_sc as plsc

VOCAB = 1000000
EMB = 32
BATCH = 16384
BAG = 50

NC, NS = 2, 16
NW = NC * NS
BAGS_PER_W = BATCH // NW
CHUNK = 64
ROWS = CHUNK * BAG
GRP = 128
NGRP = ROWS // GRP
NCHUNK = BAGS_PER_W // CHUNK
HALF = EMB // 2


def _body(idx_hbm, table_hbm, out_hbm, idx_v, rows_v, out_v, sem):
    wid = lax.axis_index("s") * NC + lax.axis_index("c")

    def chunk_body(c, carry):
        pltpu.sync_copy(idx_hbm.at[wid * NCHUNK + c], idx_v)
        copies = [
            pltpu.async_copy(
                table_hbm.at[idx_v.at[g]],
                rows_v.at[pl.ds(g * GRP, GRP)],
                sem,
            )
            for g in range(NGRP)
        ]
        for cp in copies:
            cp.wait()

        def bag_body(b, _):
            base = b * BAG

            def j_body(j, accs):
                a0, a1 = accs
                r = base + j
                a0 = a0 + rows_v[r, pl.ds(0, HALF)]
                a1 = a1 + rows_v[r, pl.ds(HALF, HALF)]
                return a0, a1

            zero = jnp.zeros((HALF,), jnp.float32)
            a0, a1 = lax.fori_loop(0, BAG, j_body, (zero, zero), unroll=BAG)
            out_v[b, pl.ds(0, HALF)] = a0
            out_v[b, pl.ds(HALF, HALF)] = a1
            return _

        lax.fori_loop(0, CHUNK, bag_body, 0)
        pltpu.sync_copy(out_v, out_hbm.at[pl.ds(wid * BAGS_PER_W + c * CHUNK, CHUNK)])
        return carry

    lax.fori_loop(0, NCHUNK, chunk_body, 0)


@jax.jit
def _run(idx_grouped, table):
    mesh = plsc.VectorSubcoreMesh(core_axis_name="c", subcore_axis_name="s")
    f = pl.kernel(
        _body,
        out_type=jax.ShapeDtypeStruct((BATCH, EMB), jnp.float32),
        mesh=mesh,
        scratch_types=[
            pltpu.VMEM((NGRP, GRP), jnp.int32),
            pltpu.VMEM((ROWS, EMB), jnp.float32),
            pltpu.VMEM((CHUNK, EMB), jnp.float32),
            pltpu.SemaphoreType.DMA,
        ],
        compiler_params=pltpu.CompilerParams(use_tc_tiling_on_sc=False),
    )
    return f(idx_grouped, table)


def kernel(indices, table):
    idx_grouped = indices.reshape(NW * NCHUNK, NGRP, GRP)
    return _run(idx_grouped, table)


# speedup vs baseline: 2.1278x; 1.0451x over previous
import functools

import jax
import jax.numpy as jnp
from jax import lax
from jax.experimental import pallas as pl
from jax.experimental.pallas import tpu as pltpu
from jax.experimental.pallas import tpu_sc as plsc

VOCAB = 1000000
EMB = 32
BATCH = 16384
BAG = 50

NC, NS = 2, 16
NW = NC * NS
BAGS_PER_W = BATCH // NW
CHUNK = 32
ROWS = CHUNK * BAG
GRP = 100
NGRP = ROWS // GRP
NCHUNK = BAGS_PER_W // CHUNK
HALF = EMB // 2


def _body(idx_hbm, table_hbm, out_hbm, idx_v, rows_v, out_v, sem0, sem1):
    wid = lax.axis_index("s") * NC + lax.axis_index("c")
    sems = (sem0, sem1)

    def stage(c, par):
        pltpu.sync_copy(idx_hbm.at[wid * NCHUNK + c], idx_v.at[par])
        for g in range(NGRP):
            pltpu.async_copy(
                table_hbm.at[idx_v.at[par].at[g]],
                rows_v.at[par].at[pl.ds(g * GRP, GRP)],
                sems[par],
            )

    def drain(par):
        pltpu.make_async_copy(
            table_hbm.at[pl.ds(0, ROWS)], rows_v.at[par], sems[par]
        ).wait()

    def accum_and_emit(c, par):
        def bag_body(b, _):
            base = b * BAG

            def j_body(j, accs):
                a0, a1 = accs
                r = base + j
                a0 = a0 + rows_v[par, r, pl.ds(0, HALF)]
                a1 = a1 + rows_v[par, r, pl.ds(HALF, HALF)]
                return a0, a1

            zero = jnp.zeros((HALF,), jnp.float32)
            a0, a1 = lax.fori_loop(0, BAG, j_body, (zero, zero), unroll=BAG)
            out_v[par, b, pl.ds(0, HALF)] = a0
            out_v[par, b, pl.ds(HALF, HALF)] = a1
            return _

        lax.fori_loop(0, CHUNK, bag_body, 0)
        pltpu.sync_copy(
            out_v.at[par],
            out_hbm.at[pl.ds(wid * BAGS_PER_W + c * CHUNK, CHUNK)],
        )

    stage(0, 0)

    def pair_body(i, carry):
        c0 = 2 * i
        c1 = c0 + 1
        stage(c1, 1)
        drain(0)
        accum_and_emit(c0, 0)

        @pl.when(c1 + 1 < NCHUNK)
        def _():
            stage(c1 + 1, 0)

        drain(1)
        accum_and_emit(c1, 1)
        return carry

    lax.fori_loop(0, NCHUNK // 2, pair_body, 0)


@jax.jit
def _run(idx_grouped, table):
    mesh = plsc.VectorSubcoreMesh(core_axis_name="c", subcore_axis_name="s")
    f = pl.kernel(
        _body,
        out_type=jax.ShapeDtypeStruct((BATCH, EMB), jnp.float32),
        mesh=mesh,
        scratch_types=[
            pltpu.VMEM((2, NGRP, GRP), jnp.int32),
            pltpu.VMEM((2, ROWS, EMB), jnp.float32),
            pltpu.VMEM((2, CHUNK, EMB), jnp.float32),
            pltpu.SemaphoreType.DMA,
            pltpu.SemaphoreType.DMA,
        ],
        compiler_params=pltpu.CompilerParams(use_tc_tiling_on_sc=False),
    )
    return f(idx_grouped, table)


def kernel(indices, table):
    idx_grouped = indices.reshape(NW * NCHUNK, NGRP, GRP)
    return _run(idx_grouped, table)
